# baseline (speedup 1.0000x reference)
import jax
import jax.numpy as jnp
from jax import lax
from jax.experimental import pallas as pl
from jax.experimental.pallas import tpu as pltpu
from jax.experimental.pallas import tpu_sc as plsc

B = 4096
D = 64
NC = 2
NS = 16
NW = NC * NS
BPW = B // NW


def _sc_body(user_hbm, item_hbm, uemb_hbm, iemb_hbm, ubias_hbm, ibias_hbm,
             urows_hbm, irows_hbm, ub_hbm, ib_hbm,
             idx_u, idx_i, urows, irows, ub, ib, sem):
    cid = lax.axis_index("c")
    sid = lax.axis_index("s")
    wid = sid * NC + cid
    base = wid * BPW

    pltpu.sync_copy(user_hbm.at[pl.ds(base, BPW)], idx_u)
    pltpu.sync_copy(item_hbm.at[pl.ds(base, BPW)], idx_i)

    cps = [
        pltpu.async_copy(uemb_hbm.at[idx_u], urows, sem),
        pltpu.async_copy(iemb_hbm.at[idx_i], irows, sem),
        pltpu.async_copy(ubias_hbm.at[idx_u], ub, sem),
        pltpu.async_copy(ibias_hbm.at[idx_i], ib, sem),
    ]
    for cp in cps:
        cp.wait()

    pltpu.sync_copy(urows, urows_hbm.at[pl.ds(base, BPW)])
    pltpu.sync_copy(irows, irows_hbm.at[pl.ds(base, BPW)])
    pltpu.sync_copy(ub, ub_hbm.at[pl.ds(base, BPW)])
    pltpu.sync_copy(ib, ib_hbm.at[pl.ds(base, BPW)])


@jax.jit
def _sc_gather(user, item, uemb, iemb, ubias, ibias):
    mesh = plsc.VectorSubcoreMesh(core_axis_name="c", subcore_axis_name="s")
    return pl.kernel(
        _sc_body,
        mesh=mesh,
        compiler_params=pltpu.CompilerParams(use_tc_tiling_on_sc=False),
        out_type=(
            jax.ShapeDtypeStruct((B, D), jnp.float32),
            jax.ShapeDtypeStruct((B, D), jnp.float32),
            jax.ShapeDtypeStruct((B,), jnp.float32),
            jax.ShapeDtypeStruct((B,), jnp.float32),
        ),
        scratch_types=[
            pltpu.VMEM((BPW,), jnp.int32),
            pltpu.VMEM((BPW,), jnp.int32),
            pltpu.VMEM((BPW, D), jnp.float32),
            pltpu.VMEM((BPW, D), jnp.float32),
            pltpu.VMEM((BPW,), jnp.float32),
            pltpu.VMEM((BPW,), jnp.float32),
            pltpu.SemaphoreType.DMA,
        ],
    )(user, item, uemb, iemb, ubias, ibias)


def _dot_body(u_ref, i_ref, o_ref):
    o_ref[...] = jnp.sum(u_ref[...] * i_ref[...], axis=1, keepdims=True)


@jax.jit
def _tc_dot(urows, irows):
    return pl.pallas_call(
        _dot_body,
        out_shape=jax.ShapeDtypeStruct((B, 1), jnp.float32),
    )(urows, irows)


def _bcast_body(ub_ref, ib_ref, r_ref, o_ref):
    o_ref[...] = jnp.maximum(ub_ref[...] + ib_ref[...] + r_ref[...], 0.0)


BM = 512


@jax.jit
def _tc_broadcast(dot_row, ub, ib):
    return pl.pallas_call(
        _bcast_body,
        grid=(B // BM,),
        in_specs=[
            pl.BlockSpec((BM, 1), lambda i: (i, 0)),
            pl.BlockSpec((BM, 1), lambda i: (i, 0)),
            pl.BlockSpec((1, B), lambda i: (0, 0)),
        ],
        out_specs=pl.BlockSpec((BM, B), lambda i: (i, 0)),
        out_shape=jax.ShapeDtypeStruct((B, B), jnp.float32),
    )(ub, ib, dot_row)


def kernel(user, item, user_emb, item_emb, user_bias_table, item_bias_table):
    urows, irows, ubg, ibg = _sc_gather(
        user, item, user_emb, item_emb,
        user_bias_table.reshape(-1), item_bias_table.reshape(-1))
    dot = _tc_dot(urows, irows)
    return _tc_broadcast(dot.reshape(1, B), ubg.reshape(B, 1), ibg.reshape(B, 1))


# speedup vs baseline: 1.2024x; 1.2024x over previous
import jax
import jax.numpy as jnp
from jax import lax
from jax.experimental import pallas as pl
from jax.experimental.pallas import tpu as pltpu
from jax.experimental.pallas import tpu_sc as plsc

B = 4096
D = 64
N = 100000
L = 16
NC = 2
NS = 16
NW = NC * NS
BPW = B // NW
KB = 8
NBLK = D // KB


def _sc_body(user_hbm, item_hbm, ut_hbm, it_hbm, ub_hbm, ib_hbm,
             dot_hbm, c_hbm,
             idx_u, idx_i, idxs_u, idxs_i,
             u0, u1, i0, i1, bub, bib, accv, cbuf, sem0, sem1, semb):
    cid = lax.axis_index("c")
    sid = lax.axis_index("s")
    wid = sid * NC + cid
    base = wid * BPW

    pltpu.sync_copy(user_hbm.at[pl.ds(base, BPW)], idx_u)
    pltpu.sync_copy(item_hbm.at[pl.ds(base, BPW)], idx_i)

    cp_ub = pltpu.async_copy(ub_hbm.at[idx_u], bub, semb)
    cp_ib = pltpu.async_copy(ib_hbm.at[idx_i], bib, semb)

    def mk_idx(d, _):
        off_u = d * N
        for c in range(BPW // L):
            s = pl.ds(c * L, L)
            idxs_u[d, s] = idx_u[s] + off_u
            idxs_i[d, s] = idx_i[s] + off_u
        return 0

    lax.fori_loop(0, D, mk_idx, 0)

    ubufs = (u0, u1)
    ibufs = (i0, i1)
    sems = (sem0, sem1)

    def fire(blk):
        p = blk % 2
        cps = []
        for k in range(KB):
            d = blk * KB + k
            cps.append(pltpu.async_copy(ut_hbm.at[idxs_u.at[d]],
                                        ubufs[p].at[k], sems[p]))
            cps.append(pltpu.async_copy(it_hbm.at[idxs_i.at[d]],
                                        ibufs[p].at[k], sems[p]))
        return cps

    def fma(blk):
        p = blk % 2
        for c in range(BPW // L):
            s = pl.ds(c * L, L)
            acc = accv[s]
            for k in range(KB):
                acc = acc + ubufs[p][k, s] * ibufs[p][k, s]
            accv[s] = acc

    for c in range(BPW // L):
        accv[pl.ds(c * L, L)] = jnp.zeros((L,), jnp.float32)

    inflight = fire(0)
    for blk in range(NBLK):
        nxt = fire(blk + 1) if blk + 1 < NBLK else []
        for cp in inflight:
            cp.wait()
        inflight = nxt
        fma(blk)

    pltpu.sync_copy(accv, dot_hbm.at[pl.ds(base, BPW)])

    cp_ub.wait()
    cp_ib.wait()
    for c in range(BPW // L):
        s = pl.ds(c * L, L)
        cbuf[s] = bub[s] + bib[s]
    pltpu.sync_copy(cbuf, c_hbm.at[pl.ds(base, BPW)])


@jax.jit
def _sc_gather_dot(user, item, ut_flat, it_flat, ub_flat, ib_flat):
    mesh = plsc.VectorSubcoreMesh(core_axis_name="c", subcore_axis_name="s")
    return pl.kernel(
        _sc_body,
        mesh=mesh,
        compiler_params=pltpu.CompilerParams(use_tc_tiling_on_sc=False),
        out_type=(
            jax.ShapeDtypeStruct((B,), jnp.float32),
            jax.ShapeDtypeStruct((B,), jnp.float32),
        ),
        scratch_types=[
            pltpu.VMEM((BPW,), jnp.int32),
            pltpu.VMEM((BPW,), jnp.int32),
            pltpu.VMEM((D, BPW), jnp.int32),
            pltpu.VMEM((D, BPW), jnp.int32),
            pltpu.VMEM((KB, BPW), jnp.float32),
            pltpu.VMEM((KB, BPW), jnp.float32),
            pltpu.VMEM((KB, BPW), jnp.float32),
            pltpu.VMEM((KB, BPW), jnp.float32),
            pltpu.VMEM((BPW,), jnp.float32),
            pltpu.VMEM((BPW,), jnp.float32),
            pltpu.VMEM((BPW,), jnp.float32),
            pltpu.VMEM((BPW,), jnp.float32),
            pltpu.SemaphoreType.DMA,
            pltpu.SemaphoreType.DMA,
            pltpu.SemaphoreType.DMA,
        ],
    )(user, item, ut_flat, it_flat, ub_flat, ib_flat)


def _bcast_body(c_ref, r_ref, o_ref):
    o_ref[...] = jnp.maximum(c_ref[...] + r_ref[...], 0.0)


BM = 512


@jax.jit
def _tc_broadcast(dot_row, c_col):
    return pl.pallas_call(
        _bcast_body,
        grid=(B // BM,),
        in_specs=[
            pl.BlockSpec((BM, 1), lambda i: (i, 0)),
            pl.BlockSpec((1, B), lambda i: (0, 0)),
        ],
        out_specs=pl.BlockSpec((BM, B), lambda i: (i, 0)),
        out_shape=jax.ShapeDtypeStruct((B, B), jnp.float32),
    )(c_col, dot_row)


def kernel(user, item, user_emb, item_emb, user_bias_table, item_bias_table):
    ut_flat = user_emb.T.reshape(-1)
    it_flat = item_emb.T.reshape(-1)
    dot, c = _sc_gather_dot(
        user, item, ut_flat, it_flat,
        user_bias_table.reshape(-1), item_bias_table.reshape(-1))
    return _tc_broadcast(dot.reshape(1, B), c.reshape(B, 1))


# speedup vs baseline: 1.6282x; 1.3542x over previous
import jax
import jax.numpy as jnp
from jax import lax
from jax.experimental import pallas as pl
from jax.experimental.pallas import tpu as pltpu
from jax.experimental.pallas import tpu_sc as plsc

B = 4096
D = 64
N = 100000
L = 16
NC = 2
NS = 16
NW = NC * NS
BPW = B // NW

UB = 4096
RG = (N + UB - 1) // UB


def _repack_body(u_ref, i_ref, o_ref):
    o_ref[:, 0:D] = jnp.transpose(u_ref[...])
    o_ref[:, D:2 * D] = jnp.transpose(i_ref[...])


@jax.jit
def _tc_repack(ut, it):
    return pl.pallas_call(
        _repack_body,
        grid=(RG,),
        in_specs=[
            pl.BlockSpec((D, UB), lambda g: (0, g)),
            pl.BlockSpec((D, UB), lambda g: (0, g)),
        ],
        out_specs=pl.BlockSpec((UB, 2 * D), lambda g: (g, 0)),
        out_shape=jax.ShapeDtypeStruct((N, 2 * D), jnp.float32),
    )(ut, it)


def _sc_body(user_hbm, item_hbm, combo_hbm, ub_hbm, ib_hbm,
             dot_hbm, c_hbm,
             idx_u, idx_i, ubuf, ibuf, bub, bib, accv, cbuf, sem, semb):
    cid = lax.axis_index("c")
    sid = lax.axis_index("s")
    wid = sid * NC + cid
    base = wid * BPW

    pltpu.sync_copy(user_hbm.at[pl.ds(base, BPW)], idx_u)
    pltpu.sync_copy(item_hbm.at[pl.ds(base, BPW)], idx_i)

    cp_ub = pltpu.async_copy(ub_hbm.at[idx_u], bub, semb)
    cp_ib = pltpu.async_copy(ib_hbm.at[idx_i], bib, semb)

    cps = [
        pltpu.async_copy(combo_hbm.at[idx_u], ubuf, sem),
        pltpu.async_copy(combo_hbm.at[idx_i], ibuf, sem),
    ]
    for cp in cps:
        cp.wait()

    lanes = lax.iota(jnp.int32, L)
    for c in range(BPW // L):
        acc = jnp.zeros((L,), jnp.float32)
        for l in range(L):
            s = c * L + l
            p = ubuf[s, pl.ds(0, L)] * ibuf[s, pl.ds(D, L)]
            for k in range(1, D // L):
                p = p + ubuf[s, pl.ds(k * L, L)] * ibuf[s, pl.ds(D + k * L, L)]
            acc = jnp.where(lanes == l, jnp.sum(p), acc)
        accv[pl.ds(c * L, L)] = acc

    pltpu.sync_copy(accv, dot_hbm.at[pl.ds(base, BPW)])

    cp_ub.wait()
    cp_ib.wait()
    for c in range(BPW // L):
        s = pl.ds(c * L, L)
        cbuf[s] = bub[s] + bib[s]
    pltpu.sync_copy(cbuf, c_hbm.at[pl.ds(base, BPW)])


@jax.jit
def _sc_gather_dot(user, item, combo, ub_flat, ib_flat):
    mesh = plsc.VectorSubcoreMesh(core_axis_name="c", subcore_axis_name="s")
    return pl.kernel(
        _sc_body,
        mesh=mesh,
        compiler_params=pltpu.CompilerParams(
            use_tc_tiling_on_sc=True, needs_layout_passes=False),
        out_type=(
            jax.ShapeDtypeStruct((B,), jnp.float32),
            jax.ShapeDtypeStruct((B,), jnp.float32),
        ),
        scratch_types=[
            pltpu.VMEM((BPW,), jnp.int32),
            pltpu.VMEM((BPW,), jnp.int32),
            pltpu.VMEM((BPW, 2 * D), jnp.float32),
            pltpu.VMEM((BPW, 2 * D), jnp.float32),
            pltpu.VMEM((BPW,), jnp.float32),
            pltpu.VMEM((BPW,), jnp.float32),
            pltpu.VMEM((BPW,), jnp.float32),
            pltpu.VMEM((BPW,), jnp.float32),
            pltpu.SemaphoreType.DMA,
            pltpu.SemaphoreType.DMA,
        ],
    )(user, item, combo, ub_flat, ib_flat)


def _bcast_body(c_ref, r_ref, o_ref):
    o_ref[...] = jnp.maximum(c_ref[...] + r_ref[...], 0.0)


BM = 512


@jax.jit
def _tc_broadcast(dot_row, c_col):
    return pl.pallas_call(
        _bcast_body,
        grid=(B // BM,),
        in_specs=[
            pl.BlockSpec((BM, 1), lambda i: (i, 0)),
            pl.BlockSpec((1, B), lambda i: (0, 0)),
        ],
        out_specs=pl.BlockSpec((BM, B), lambda i: (i, 0)),
        out_shape=jax.ShapeDtypeStruct((B, B), jnp.float32),
    )(c_col, dot_row)


def kernel(user, item, user_emb, item_emb, user_bias_table, item_bias_table):
    combo = _tc_repack(user_emb.T, item_emb.T)
    dot, c = _sc_gather_dot(user, item, combo,
                            user_bias_table.reshape(-1),
                            item_bias_table.reshape(-1))
    return _tc_broadcast(dot.reshape(1, B), c.reshape(B, 1))
